# Initial kernel scaffold; baseline (speedup 1.0000x reference)
import functools

import jax
import jax.numpy as jnp
from jax import lax
from jax.experimental import pallas as pl
from jax.experimental.pallas import tpu as pltpu
from jax.experimental.pallas import tpu_sc as plsc

N_NODES = 10000
N_PAD = 10240
N_EDGES = 320000
FEAT = 128

NC, NS, L = 2, 16, 16
NW = NC * NS

_mesh = functools.partial(
    plsc.VectorSubcoreMesh, core_axis_name="c", subcore_axis_name="s")


_EPT_A = N_EDGES // NS
_HALF = N_PAD // NC


@functools.partial(
    pl.kernel,
    out_type=jax.ShapeDtypeStruct((N_PAD,), jnp.int32),
    mesh=_mesh(),
    scratch_types=[
        pltpu.VMEM((_EPT_A,), jnp.int32),
        pltpu.VMEM((N_PAD,), jnp.int32),
        pltpu.VMEM((_HALF // NS,), jnp.int32),
        pltpu.VMEM_SHARED((NS, N_PAD), jnp.int32),
    ],
)
def _deg_kernel(edge_hbm, cnt_hbm, dst_v, acc_v, red_v, shared):
    cid = lax.axis_index("c")
    sid = lax.axis_index("s")
    zeros = jnp.zeros((L,), jnp.int32)

    def zero_body(i, _):
        acc_v[pl.ds(i * L, L)] = zeros

    lax.fori_loop(0, N_PAD // L, zero_body, None, unroll=8)

    pltpu.sync_copy(edge_hbm.at[1, pl.ds(sid * _EPT_A, _EPT_A)], dst_v)
    ones = jnp.ones((L,), jnp.int32)

    def body(i, _):
        idx = dst_v[pl.ds(i * L, L)]
        plsc.addupdate_scatter(acc_v, [idx], ones)

    lax.fori_loop(0, _EPT_A // L, body, None, unroll=4)

    pltpu.sync_copy(acc_v, shared.at[sid])
    plsc.subcore_barrier()

    chunk = _HALF // NS
    base = cid * _HALF + sid * chunk

    def zero_red(i, _):
        red_v[pl.ds(i * L, L)] = zeros

    lax.fori_loop(0, chunk // L, zero_red, None, unroll=8)

    def red_body(t, _):
        pltpu.sync_copy(shared.at[t, pl.ds(base, chunk)], dst_v.at[pl.ds(0, chunk)])

        def add_body(i, _):
            red_v[pl.ds(i * L, L)] = red_v[pl.ds(i * L, L)] + dst_v[pl.ds(i * L, L)]

        lax.fori_loop(0, chunk // L, add_body, None, unroll=8)

    lax.fori_loop(0, NS, red_body, None)
    pltpu.sync_copy(red_v, cnt_hbm.at[pl.ds(base, chunk)])


_EPC = N_EDGES // NC
_EPT_B = _EPC // NS
_BB = 80
_NBATCH = _EPT_B // _BB
_ROWS_PER_TILE = N_PAD // NS


@functools.partial(
    pl.kernel,
    out_type=jax.ShapeDtypeStruct((NC, N_PAD, FEAT), jnp.float32),
    mesh=_mesh(),
    scratch_types=[
        pltpu.VMEM((_BB,), jnp.int32),
        pltpu.VMEM((_BB,), jnp.int32),
        pltpu.VMEM((_BB, FEAT), jnp.float32),
        pltpu.VMEM((_BB, FEAT), jnp.float32),
        pltpu.VMEM_SHARED((N_PAD, FEAT), jnp.float32),
        pltpu.SemaphoreType.DMA,
    ],
)
def _msg_kernel(edge_hbm, p_hbm, out_hbm, src_v, dst_v, rows_v, zblk_v, acc, sem):
    cid = lax.axis_index("c")
    sid = lax.axis_index("s")
    zeros = jnp.zeros((L,), jnp.float32)

    def zero_body(i, _):
        zblk_v[i // (FEAT // L), pl.ds((i % (FEAT // L)) * L, L)] = zeros

    lax.fori_loop(0, _BB * FEAT // L, zero_body, None, unroll=8)

    row0 = sid * _ROWS_PER_TILE
    for j in range(_ROWS_PER_TILE // _BB):
        pltpu.sync_copy(zblk_v, acc.at[pl.ds(row0 + j * _BB, _BB)])
    plsc.subcore_barrier()

    ebase = cid * _EPC + sid * _EPT_B

    def body(i, _):
        b = ebase + i * _BB
        pltpu.sync_copy(edge_hbm.at[0, pl.ds(b, _BB)], src_v)
        pltpu.sync_copy(edge_hbm.at[1, pl.ds(b, _BB)], dst_v)
        pltpu.async_copy(p_hbm.at[src_v], rows_v, sem).wait()
        pltpu.sync_copy(rows_v, acc.at[dst_v], add=True)

    lax.fori_loop(0, _NBATCH, body, None)
    plsc.subcore_barrier()
    pltpu.sync_copy(acc.at[pl.ds(row0, _ROWS_PER_TILE)],
                    out_hbm.at[cid, pl.ds(row0, _ROWS_PER_TILE)])


_EPT_C = N_EDGES // NS


@functools.partial(
    pl.kernel,
    out_type=jax.ShapeDtypeStruct((N_PAD,), jnp.float32),
    mesh=_mesh(),
    scratch_types=[
        pltpu.VMEM((_EPT_C,), jnp.int32),
        pltpu.VMEM((_EPT_C,), jnp.int32),
        pltpu.VMEM((N_PAD,), jnp.float32),
        pltpu.VMEM((N_PAD,), jnp.float32),
        pltpu.VMEM((_HALF // NS,), jnp.float32),
        pltpu.VMEM((_HALF // NS,), jnp.float32),
        pltpu.VMEM_SHARED((NS, N_PAD), jnp.float32),
    ],
)
def _scalar_kernel(edge_hbm, q_hbm, dinv_hbm, r_hbm, out_hbm,
                   src_v, dst_v, q_v, acc_v, red_v, aux_v, shared):
    cid = lax.axis_index("c")
    sid = lax.axis_index("s")
    zeros = jnp.zeros((L,), jnp.float32)

    def zero_body(i, _):
        acc_v[pl.ds(i * L, L)] = zeros

    lax.fori_loop(0, N_PAD // L, zero_body, None, unroll=8)

    pltpu.sync_copy(q_hbm, q_v)
    pltpu.sync_copy(edge_hbm.at[0, pl.ds(sid * _EPT_C, _EPT_C)], src_v)
    pltpu.sync_copy(edge_hbm.at[1, pl.ds(sid * _EPT_C, _EPT_C)], dst_v)

    def body(i, _):
        s_idx = src_v[pl.ds(i * L, L)]
        d_idx = dst_v[pl.ds(i * L, L)]
        vals = plsc.load_gather(q_v, [s_idx])
        plsc.addupdate_scatter(acc_v, [d_idx], vals)

    lax.fori_loop(0, _EPT_C // L, body, None, unroll=4)

    pltpu.sync_copy(acc_v, shared.at[sid])
    plsc.subcore_barrier()

    chunk = _HALF // NS
    base = cid * _HALF + sid * chunk

    def zero_red(i, _):
        red_v[pl.ds(i * L, L)] = zeros

    lax.fori_loop(0, chunk // L, zero_red, None, unroll=8)

    def red_body(t, _):
        pltpu.sync_copy(shared.at[t, pl.ds(base, chunk)], acc_v.at[pl.ds(0, chunk)])

        def add_body(i, _):
            red_v[pl.ds(i * L, L)] = red_v[pl.ds(i * L, L)] + acc_v[pl.ds(i * L, L)]

        lax.fori_loop(0, chunk // L, add_body, None, unroll=8)

    lax.fori_loop(0, NS, red_body, None)

    pltpu.sync_copy(dinv_hbm.at[pl.ds(base, chunk)], aux_v)

    def mul_body(i, _):
        red_v[pl.ds(i * L, L)] = red_v[pl.ds(i * L, L)] * aux_v[pl.ds(i * L, L)]

    lax.fori_loop(0, chunk // L, mul_body, None, unroll=8)
    pltpu.sync_copy(r_hbm.at[pl.ds(base, chunk)], aux_v)

    def add_r_body(i, _):
        red_v[pl.ds(i * L, L)] = red_v[pl.ds(i * L, L)] + aux_v[pl.ds(i * L, L)]

    lax.fori_loop(0, chunk // L, add_r_body, None, unroll=8)
    pltpu.sync_copy(red_v, out_hbm.at[pl.ds(base, chunk)])


_RB = 2048
_NRB = N_PAD // _RB


def _tc1_body(cnt_ref, x_ref, w1_ref, dinv_ref, p_ref):
    deg = cnt_ref[...].astype(jnp.float32) + 1.0
    dinv = lax.rsqrt(deg)
    dinv_ref[...] = dinv
    h = jnp.dot(x_ref[...], w1_ref[...], preferred_element_type=jnp.float32)
    p_ref[...] = dinv * h


def _tc1(cnt, x_pad, W1):
    return pl.pallas_call(
        _tc1_body,
        grid=(_NRB,),
        in_specs=[
            pl.BlockSpec((_RB, 1), lambda i: (i, 0)),
            pl.BlockSpec((_RB, FEAT), lambda i: (i, 0)),
            pl.BlockSpec((FEAT, FEAT), lambda i: (0, 0)),
        ],
        out_specs=[
            pl.BlockSpec((_RB, 1), lambda i: (i, 0)),
            pl.BlockSpec((_RB, FEAT), lambda i: (i, 0)),
        ],
        out_shape=[
            jax.ShapeDtypeStruct((N_PAD, 1), jnp.float32),
            jax.ShapeDtypeStruct((N_PAD, FEAT), jnp.float32),
        ],
    )(cnt, x_pad, W1)


def _tc2_body(s_ref, p_ref, dinv_ref, b1_ref, w2_ref, b2_ref, q_ref, r_ref):
    dinv = dinv_ref[...]
    s = s_ref[0] + s_ref[1] + p_ref[...]
    t = jnp.maximum(dinv * s + b1_ref[...], 0.0)
    z = jnp.dot(t, w2_ref[...], preferred_element_type=jnp.float32)
    q = dinv * z
    q_ref[...] = q
    r_ref[...] = dinv * q + b2_ref[0, 0]


def _tc2(s_parts, p, dinv, b1, W2, b2):
    return pl.pallas_call(
        _tc2_body,
        grid=(_NRB,),
        in_specs=[
            pl.BlockSpec((NC, _RB, FEAT), lambda i: (0, i, 0)),
            pl.BlockSpec((_RB, FEAT), lambda i: (i, 0)),
            pl.BlockSpec((_RB, 1), lambda i: (i, 0)),
            pl.BlockSpec((1, FEAT), lambda i: (0, 0)),
            pl.BlockSpec((FEAT, 1), lambda i: (0, 0)),
            pl.BlockSpec((1, 1), lambda i: (0, 0)),
        ],
        out_specs=[
            pl.BlockSpec((_RB, 1), lambda i: (i, 0)),
            pl.BlockSpec((_RB, 1), lambda i: (i, 0)),
        ],
        out_shape=[
            jax.ShapeDtypeStruct((N_PAD, 1), jnp.float32),
            jax.ShapeDtypeStruct((N_PAD, 1), jnp.float32),
        ],
    )(s_parts, p, dinv, b1, W2, b2)


def kernel(x, edge_index, W1, b1, W2, b2):
    edge_index = edge_index.astype(jnp.int32)
    x_pad = jnp.pad(x, ((0, N_PAD - N_NODES), (0, 0)))

    cnt = _deg_kernel(edge_index)
    dinv, p = _tc1(cnt.reshape(N_PAD, 1), x_pad, W1)
    s_parts = _msg_kernel(edge_index, p)
    q, r = _tc2(s_parts, p, dinv, b1.reshape(1, FEAT), W2, b2.reshape(1, 1))
    out = _scalar_kernel(edge_index, q.reshape(-1), dinv.reshape(-1),
                         r.reshape(-1))
    return out[:N_NODES]


# speedup vs baseline: 25.8272x; 25.8272x over previous
import functools

import jax
import jax.numpy as jnp
from jax import lax
from jax.experimental import pallas as pl
from jax.experimental.pallas import tpu as pltpu
from jax.experimental.pallas import tpu_sc as plsc

N_NODES = 10000
N_PAD = 10240
N_EDGES = 320000
FEAT = 128

NC, NS, L = 2, 16, 16
NW = NC * NS

_mesh = functools.partial(
    plsc.VectorSubcoreMesh, core_axis_name="c", subcore_axis_name="s")


_EPT_A = N_EDGES // NS
_HALF = N_PAD // NC


@functools.partial(
    pl.kernel,
    out_type=jax.ShapeDtypeStruct((N_PAD,), jnp.int32),
    mesh=_mesh(),
    compiler_params=pltpu.CompilerParams(needs_layout_passes=False),
    scratch_types=[
        pltpu.VMEM((_EPT_A,), jnp.int32),
        pltpu.VMEM((N_PAD,), jnp.int32),
        pltpu.VMEM((_HALF // NS,), jnp.int32),
        pltpu.VMEM_SHARED((NS * N_PAD,), jnp.int32),
    ],
)
def _deg_kernel(dst_hbm, cnt_hbm, dst_v, acc_v, red_v, shared):
    cid = lax.axis_index("c")
    sid = lax.axis_index("s")
    zeros = jnp.zeros((L,), jnp.int32)

    def zero_body(i, _):
        acc_v[pl.ds(i * L, L)] = zeros

    lax.fori_loop(0, N_PAD // L, zero_body, None, unroll=8)

    pltpu.sync_copy(dst_hbm.at[pl.ds(sid * _EPT_A, _EPT_A)], dst_v)
    ones = jnp.ones((L,), jnp.int32)

    def body(i, _):
        idx = dst_v[pl.ds(i * L, L)]
        plsc.addupdate_scatter(acc_v, [idx], ones)

    lax.fori_loop(0, _EPT_A // L, body, None, unroll=4)

    pltpu.sync_copy(acc_v, shared.at[pl.ds(sid * N_PAD, N_PAD)])
    plsc.subcore_barrier()

    chunk = _HALF // NS
    base = cid * _HALF + sid * chunk

    def zero_red(i, _):
        red_v[pl.ds(i * L, L)] = zeros

    lax.fori_loop(0, chunk // L, zero_red, None, unroll=8)

    def red_body(t, _):
        pltpu.sync_copy(shared.at[pl.ds(t * N_PAD + base, chunk)],
                        dst_v.at[pl.ds(0, chunk)])

        def add_body(i, _):
            red_v[pl.ds(i * L, L)] = red_v[pl.ds(i * L, L)] + dst_v[pl.ds(i * L, L)]

        lax.fori_loop(0, chunk // L, add_body, None, unroll=8)

    lax.fori_loop(0, NS, red_body, None)
    pltpu.sync_copy(red_v, cnt_hbm.at[pl.ds(base, chunk)])


_EPC = N_EDGES // NC
_EPT_B = _EPC // NS
_BB = 80
_NBATCH = _EPT_B // _BB
_ROWS_PER_TILE = N_PAD // NS


@functools.partial(
    pl.kernel,
    out_type=jax.ShapeDtypeStruct((NC, N_PAD, FEAT), jnp.float32),
    mesh=_mesh(),
    compiler_params=pltpu.CompilerParams(needs_layout_passes=False),
    scratch_types=[
        pltpu.VMEM((_BB,), jnp.int32),
        pltpu.VMEM((_BB,), jnp.int32),
        pltpu.VMEM((_BB, FEAT), jnp.float32),
        pltpu.VMEM((_BB, FEAT), jnp.float32),
        pltpu.VMEM_SHARED((N_PAD, FEAT), jnp.float32),
        pltpu.SemaphoreType.DMA,
    ],
)
def _msg_kernel(src_hbm, dst_hbm, p_hbm, out_hbm, src_v, dst_v, rows_v, zblk_v,
                acc, sem):
    cid = lax.axis_index("c")
    sid = lax.axis_index("s")
    zeros = jnp.zeros((L,), jnp.float32)

    def zero_body(i, _):
        zblk_v[i // (FEAT // L), pl.ds((i % (FEAT // L)) * L, L)] = zeros

    lax.fori_loop(0, _BB * FEAT // L, zero_body, None, unroll=8)

    row0 = sid * _ROWS_PER_TILE
    for j in range(_ROWS_PER_TILE // _BB):
        pltpu.sync_copy(zblk_v, acc.at[pl.ds(row0 + j * _BB, _BB)])
    plsc.subcore_barrier()

    ebase = cid * _EPC + sid * _EPT_B

    def body(i, _):
        b = ebase + i * _BB
        pltpu.sync_copy(src_hbm.at[pl.ds(b, _BB)], src_v)
        pltpu.sync_copy(dst_hbm.at[pl.ds(b, _BB)], dst_v)
        pltpu.async_copy(p_hbm.at[src_v], rows_v, sem).wait()
        pltpu.sync_copy(rows_v, acc.at[dst_v], add=True)

    lax.fori_loop(0, _NBATCH, body, None)
    plsc.subcore_barrier()
    pltpu.sync_copy(acc.at[pl.ds(row0, _ROWS_PER_TILE)],
                    out_hbm.at[cid, pl.ds(row0, _ROWS_PER_TILE)])


_EPT_C = N_EDGES // NS


@functools.partial(
    pl.kernel,
    out_type=jax.ShapeDtypeStruct((N_PAD,), jnp.float32),
    mesh=_mesh(),
    compiler_params=pltpu.CompilerParams(needs_layout_passes=False),
    scratch_types=[
        pltpu.VMEM((_EPT_C,), jnp.int32),
        pltpu.VMEM((_EPT_C,), jnp.int32),
        pltpu.VMEM((N_PAD,), jnp.float32),
        pltpu.VMEM((N_PAD,), jnp.float32),
        pltpu.VMEM((_HALF // NS,), jnp.float32),
        pltpu.VMEM((_HALF // NS,), jnp.float32),
        pltpu.VMEM_SHARED((NS * N_PAD,), jnp.float32),
    ],
)
def _scalar_kernel(src_hbm, dst_hbm, q_hbm, dinv_hbm, r_hbm, out_hbm,
                   src_v, dst_v, q_v, acc_v, red_v, aux_v, shared):
    cid = lax.axis_index("c")
    sid = lax.axis_index("s")
    zeros = jnp.zeros((L,), jnp.float32)

    def zero_body(i, _):
        acc_v[pl.ds(i * L, L)] = zeros

    lax.fori_loop(0, N_PAD // L, zero_body, None, unroll=8)

    pltpu.sync_copy(q_hbm, q_v)
    pltpu.sync_copy(src_hbm.at[pl.ds(sid * _EPT_C, _EPT_C)], src_v)
    pltpu.sync_copy(dst_hbm.at[pl.ds(sid * _EPT_C, _EPT_C)], dst_v)

    def body(i, _):
        s_idx = src_v[pl.ds(i * L, L)]
        d_idx = dst_v[pl.ds(i * L, L)]
        vals = plsc.load_gather(q_v, [s_idx])
        plsc.addupdate_scatter(acc_v, [d_idx], vals)

    lax.fori_loop(0, _EPT_C // L, body, None, unroll=4)

    pltpu.sync_copy(acc_v, shared.at[pl.ds(sid * N_PAD, N_PAD)])
    plsc.subcore_barrier()

    chunk = _HALF // NS
    base = cid * _HALF + sid * chunk

    def zero_red(i, _):
        red_v[pl.ds(i * L, L)] = zeros

    lax.fori_loop(0, chunk // L, zero_red, None, unroll=8)

    def red_body(t, _):
        pltpu.sync_copy(shared.at[pl.ds(t * N_PAD + base, chunk)],
                        acc_v.at[pl.ds(0, chunk)])

        def add_body(i, _):
            red_v[pl.ds(i * L, L)] = red_v[pl.ds(i * L, L)] + acc_v[pl.ds(i * L, L)]

        lax.fori_loop(0, chunk // L, add_body, None, unroll=8)

    lax.fori_loop(0, NS, red_body, None)

    pltpu.sync_copy(dinv_hbm.at[pl.ds(base, chunk)], aux_v)

    def mul_body(i, _):
        red_v[pl.ds(i * L, L)] = red_v[pl.ds(i * L, L)] * aux_v[pl.ds(i * L, L)]

    lax.fori_loop(0, chunk // L, mul_body, None, unroll=8)
    pltpu.sync_copy(r_hbm.at[pl.ds(base, chunk)], aux_v)

    def add_r_body(i, _):
        red_v[pl.ds(i * L, L)] = red_v[pl.ds(i * L, L)] + aux_v[pl.ds(i * L, L)]

    lax.fori_loop(0, chunk // L, add_r_body, None, unroll=8)
    pltpu.sync_copy(red_v, out_hbm.at[pl.ds(base, chunk)])


_RB = 2048
_NRB = N_PAD // _RB


def _tc1_body(cnt_ref, x_ref, w1_ref, dinv_ref, p_ref):
    deg = cnt_ref[...].astype(jnp.float32) + 1.0
    dinv = lax.rsqrt(deg)
    dinv_ref[...] = dinv
    h = jnp.dot(x_ref[...], w1_ref[...], preferred_element_type=jnp.float32)
    p_ref[...] = dinv * h


def _tc1(cnt, x_pad, W1):
    return pl.pallas_call(
        _tc1_body,
        grid=(_NRB,),
        in_specs=[
            pl.BlockSpec((_RB, 1), lambda i: (i, 0)),
            pl.BlockSpec((_RB, FEAT), lambda i: (i, 0)),
            pl.BlockSpec((FEAT, FEAT), lambda i: (0, 0)),
        ],
        out_specs=[
            pl.BlockSpec((_RB, 1), lambda i: (i, 0)),
            pl.BlockSpec((_RB, FEAT), lambda i: (i, 0)),
        ],
        out_shape=[
            jax.ShapeDtypeStruct((N_PAD, 1), jnp.float32),
            jax.ShapeDtypeStruct((N_PAD, FEAT), jnp.float32),
        ],
    )(cnt, x_pad, W1)


def _tc2_body(s_ref, p_ref, dinv_ref, b1_ref, w2_ref, b2_ref, q_ref, r_ref):
    dinv = dinv_ref[...]
    s = s_ref[0] + s_ref[1] + p_ref[...]
    t = jnp.maximum(dinv * s + b1_ref[...], 0.0)
    z = jnp.dot(t, w2_ref[...], preferred_element_type=jnp.float32)
    q = dinv * z
    q_ref[...] = q
    r_ref[...] = dinv * q + b2_ref[0, 0]


def _tc2(s_parts, p, dinv, b1, W2, b2):
    return pl.pallas_call(
        _tc2_body,
        grid=(_NRB,),
        in_specs=[
            pl.BlockSpec((NC, _RB, FEAT), lambda i: (0, i, 0)),
            pl.BlockSpec((_RB, FEAT), lambda i: (i, 0)),
            pl.BlockSpec((_RB, 1), lambda i: (i, 0)),
            pl.BlockSpec((1, FEAT), lambda i: (0, 0)),
            pl.BlockSpec((FEAT, 1), lambda i: (0, 0)),
            pl.BlockSpec((1, 1), lambda i: (0, 0)),
        ],
        out_specs=[
            pl.BlockSpec((_RB, 1), lambda i: (i, 0)),
            pl.BlockSpec((_RB, 1), lambda i: (i, 0)),
        ],
        out_shape=[
            jax.ShapeDtypeStruct((N_PAD, 1), jnp.float32),
            jax.ShapeDtypeStruct((N_PAD, 1), jnp.float32),
        ],
    )(s_parts, p, dinv, b1, W2, b2)


def kernel(x, edge_index, W1, b1, W2, b2):
    edge_index = edge_index.astype(jnp.int32)
    src_idx = edge_index[0]
    dst_idx = edge_index[1]
    x_pad = jnp.pad(x, ((0, N_PAD - N_NODES), (0, 0)))

    cnt = _deg_kernel(dst_idx)
    dinv, p = _tc1(cnt.reshape(N_PAD, 1), x_pad, W1)
    s_parts = _msg_kernel(src_idx, dst_idx, p)
    q, r = _tc2(s_parts, p, dinv, b1.reshape(1, FEAT), W2, b2.reshape(1, 1))
    out = _scalar_kernel(src_idx, dst_idx, q.reshape(-1), dinv.reshape(-1),
                         r.reshape(-1))
    return out[:N_NODES]
